# baseline (speedup 1.0000x reference)
import functools

import jax
import jax.numpy as jnp
from jax import lax
from jax.experimental import pallas as pl
from jax.experimental.pallas import tpu as pltpu
from jax.experimental.pallas import tpu_sc as plsc

D = 64
VOCAB = 1000000
NW = 32
NC = 2
VT_FULL = VOCAB // 128
VT_TAIL = VOCAB - VT_FULL * 128
NT = 200

_MESH = plsc.VectorSubcoreMesh(core_axis_name="c", subcore_axis_name="s")
_PARAMS = pltpu.CompilerParams(
    use_tc_tiling_on_sc=True, needs_layout_passes=False
)


def _iota16():
  return lax.iota(jnp.int32, 16)


@functools.partial(
    pl.kernel,
    mesh=_MESH,
    out_type=jax.ShapeDtypeStruct((VOCAB // 2, 128), jnp.float32),
    scratch_types=[
        pltpu.VMEM((2, D, 128), jnp.float32),
        pltpu.VMEM((2, D, 128), jnp.float32),
        pltpu.SemaphoreType.DMA,
        pltpu.SemaphoreType.DMA,
        pltpu.SemaphoreType.DMA,
        pltpu.SemaphoreType.DMA,
    ],
    compiler_params=_PARAMS,
)
def _relayout(table_t, pairs, tin, tout, gi0, gi1, go0, go1):
  wid = lax.axis_index("s") * NC + lax.axis_index("c")
  n_steps = 244 + (wid < VT_FULL - 244 * NW).astype(jnp.int32)
  isems = [gi0, gi1]
  osems = [go0, go1]

  rowd = [_iota16() + dg * 16 for dg in range(4)]
  perms = [jnp.bitwise_and(_iota16() + k, 15) for k in range(16)]
  cpart = [
      lax.shift_left(jnp.bitwise_and(p, 1), 6) + _iota16() for p in perms
  ]

  def start_in(i, half):
    vt = i * NW + wid
    pltpu.async_copy(
        table_t.at[:, pl.ds(vt * 128, 128)], tin.at[half], isems[half]
    )

  def drain_in(half):
    pltpu.make_async_copy(
        table_t.at[:, pl.ds(0, 128)], tin.at[half], isems[half]
    ).wait()

  def drain_out(half):
    pltpu.make_async_copy(
        tout.at[half], pairs.at[pl.ds(0, 64), :], osems[half]
    ).wait()

  def transpose_tile(half):
    src = tin.at[half]
    dst = tout.at[half]

    @plsc.parallel_loop(0, 8, step=1, unroll=4)
    def _(vg):
      vl0 = vg * 16
      for k in range(16):
        colv = perms[k] + vl0
        rv = lax.shift_right_logical(colv, 1)
        vecs = [plsc.load_gather(src, [rowd[dg], colv]) for dg in range(4)]
        for dg in range(4):
          cv = cpart[k] + dg * 16
          plsc.store_scatter(dst, [rv, cv], vecs[dg])

  start_in(0, 0)
  start_in(1, 1)

  def pair_step(p, carry):
    for half in range(2):
      i = 2 * p + half

      @pl.when(i < n_steps)
      def _():
        drain_in(half)

        @pl.when(i >= 2)
        def _():
          drain_out(half)

        transpose_tile(half)
        vt = i * NW + wid
        pltpu.async_copy(
            tout.at[half], pairs.at[pl.ds(vt * 64, 64), :], osems[half]
        )

        @pl.when(i + 2 < n_steps)
        def _():
          start_in(i + 2, half)

    return carry

  lax.fori_loop(0, 123, pair_step, 0)
  drain_out(0)
  drain_out(1)


@functools.partial(
    pl.kernel,
    mesh=_MESH,
    out_type=jax.ShapeDtypeStruct((NT, D, 4096), jnp.float32),
    scratch_types=[
        pltpu.VMEM((NT, 128), jnp.int32),
        pltpu.VMEM((3, 128), jnp.int32),
        pltpu.VMEM((3, 128), jnp.int32),
        pltpu.VMEM((3, 128, 128), jnp.float32),
        pltpu.VMEM((2, D, 128), jnp.float32),
        pltpu.SemaphoreType.DMA,
        pltpu.SemaphoreType.DMA,
        pltpu.SemaphoreType.DMA,
        pltpu.SemaphoreType.DMA,
        pltpu.SemaphoreType.DMA,
    ],
    compiler_params=_PARAMS,
)
def _gather(x_t, pairs, out, idx_all, gidx, hbuf, prows, tout, gg0, gg1, gg2,
            go0, go1):
  wid = lax.axis_index("s") * NC + lax.axis_index("c")
  gsems = [gg0, gg1, gg2]
  osems = [go0, go1]
  rowd = [_iota16() + dg * 16 for dg in range(4)]
  perms = [jnp.bitwise_and(_iota16() + k, 15) for k in range(16)]

  pltpu.sync_copy(x_t.at[:, pl.ds(wid * 128, 128)], idx_all)

  def prep_and_fire(t, buf):
    for k in range(8):
      v = idx_all[t, pl.ds(k * 16, 16)]
      gidx[buf, pl.ds(k * 16, 16)] = lax.shift_right_logical(v, 1)
      hbuf[buf, pl.ds(k * 16, 16)] = lax.shift_left(
          jnp.bitwise_and(v, 1), 6
      )
    pltpu.async_copy(pairs.at[gidx.at[buf]], prows.at[buf], gsems[buf])

  def drain_gather(buf):
    pltpu.make_async_copy(
        pairs.at[gidx.at[buf]], prows.at[buf], gsems[buf]
    ).wait()

  def drain_out(buf):
    pltpu.make_async_copy(
        tout.at[buf], out.at[0, :, pl.ds(0, 128)], osems[buf]
    ).wait()

  def transpose_block(buf, ob):
    src = prows.at[buf]
    dst = tout.at[ob]
    hsrc = hbuf.at[buf]

    @plsc.parallel_loop(0, 8, step=1, unroll=4)
    def _(bg):
      b0 = bg * 16
      for k in range(16):
        cold = perms[k] + b0
        halfp = plsc.load_gather(hsrc, [cold])
        vecs = [
            plsc.load_gather(src, [cold, halfp + rowd[dg]])
            for dg in range(4)
        ]
        for dg in range(4):
          plsc.store_scatter(dst, [rowd[dg], cold], vecs[dg])

  prep_and_fire(0, 0)
  prep_and_fire(1, 1)

  def sextet_step(p, carry):
    for j in range(6):
      i = 6 * p + j
      gbuf = j % 3
      obuf = j % 2

      @pl.when(i < NT)
      def _():
        @pl.when(i + 2 < NT)
        def _():
          prep_and_fire(i + 2, (j + 2) % 3)

        drain_gather(gbuf)

        @pl.when(i >= 2)
        def _():
          drain_out(obuf)

        transpose_block(gbuf, obuf)
        pltpu.async_copy(
            tout.at[obuf], out.at[i, :, pl.ds(wid * 128, 128)], osems[obuf]
        )

    return carry

  lax.fori_loop(0, (NT + 5) // 6, sextet_step, 0)
  drain_out(0)
  drain_out(1)


@jax.jit
def kernel(x, table):
  B0, T = x.shape
  xt = jnp.asarray(x, jnp.int32).T
  tt = table.T
  pairs = _relayout(tt)
  tail = table[VT_FULL * 128 :, :].reshape(VT_TAIL // 2, 128)
  pairs = lax.dynamic_update_slice(pairs, tail, (VT_FULL * 64, 0))
  out5 = _gather(xt, pairs)
  return out5.transpose(2, 0, 1)


# speedup vs baseline: 1.0345x; 1.0345x over previous
import functools

import jax
import jax.numpy as jnp
from jax import lax
from jax.experimental import pallas as pl
from jax.experimental.pallas import tpu as pltpu
from jax.experimental.pallas import tpu_sc as plsc

D = 64
VOCAB = 1000000
NW = 32
NC = 2
VT_FULL = VOCAB // 128
VT_TAIL = VOCAB - VT_FULL * 128
NT = 200

_MESH = plsc.VectorSubcoreMesh(core_axis_name="c", subcore_axis_name="s")
_PARAMS = pltpu.CompilerParams(
    use_tc_tiling_on_sc=True, needs_layout_passes=False
)


def _iota16():
  return lax.iota(jnp.int32, 16)


@functools.partial(
    pl.kernel,
    mesh=_MESH,
    out_type=jax.ShapeDtypeStruct((VOCAB // 2, 128), jnp.float32),
    scratch_types=[
        pltpu.VMEM((2, D, 128), jnp.float32),
        pltpu.VMEM((2, D, 128), jnp.float32),
        pltpu.SemaphoreType.DMA,
        pltpu.SemaphoreType.DMA,
        pltpu.SemaphoreType.DMA,
        pltpu.SemaphoreType.DMA,
    ],
    compiler_params=_PARAMS,
)
def _relayout(table_t, pairs, tin, tout, gi0, gi1, go0, go1):
  wid = lax.axis_index("s") * NC + lax.axis_index("c")
  n_steps = 244 + (wid < VT_FULL - 244 * NW).astype(jnp.int32)
  isems = [gi0, gi1]
  osems = [go0, go1]

  rowd = [_iota16() + dg * 16 for dg in range(4)]
  perms = [jnp.bitwise_and(_iota16() + k, 15) for k in range(16)]
  cpart = [
      lax.shift_left(jnp.bitwise_and(p, 1), 6) + _iota16() for p in perms
  ]

  def start_in(i, half):
    vt = i * NW + wid
    pltpu.async_copy(
        table_t.at[:, pl.ds(vt * 128, 128)], tin.at[half], isems[half]
    )

  def drain_in(half):
    pltpu.make_async_copy(
        table_t.at[:, pl.ds(0, 128)], tin.at[half], isems[half]
    ).wait()

  def drain_out(half):
    pltpu.make_async_copy(
        tout.at[half], pairs.at[pl.ds(0, 64), :], osems[half]
    ).wait()

  def transpose_tile(half):
    src = tin.at[half]
    dst = tout.at[half]

    @plsc.parallel_loop(0, 8, step=1, unroll=4)
    def _(vg):
      vl0 = vg * 16
      for k in range(16):
        colv = perms[k] + vl0
        rv = lax.shift_right_logical(colv, 1)
        vecs = [plsc.load_gather(src, [rowd[dg], colv]) for dg in range(4)]
        for dg in range(4):
          cv = cpart[k] + dg * 16
          plsc.store_scatter(dst, [rv, cv], vecs[dg])

  start_in(0, 0)
  start_in(1, 1)

  def pair_step(p, carry):
    for half in range(2):
      i = 2 * p + half

      @pl.when(i < n_steps)
      def _():
        drain_in(half)

        @pl.when(i >= 2)
        def _():
          drain_out(half)

        transpose_tile(half)
        vt = i * NW + wid
        pltpu.async_copy(
            tout.at[half], pairs.at[pl.ds(vt * 64, 64), :], osems[half]
        )

        @pl.when(i + 2 < n_steps)
        def _():
          start_in(i + 2, half)

    return carry

  lax.fori_loop(0, 123, pair_step, 0)
  drain_out(0)
  drain_out(1)


@functools.partial(
    pl.kernel,
    mesh=_MESH,
    out_type=jax.ShapeDtypeStruct((NT, D, 4096), jnp.float32),
    scratch_types=[
        pltpu.VMEM((NT, 128), jnp.int32),
        pltpu.VMEM((2, 128), jnp.int32),
        pltpu.VMEM((2, 128), jnp.int32),
        pltpu.VMEM((2, 128, 128), jnp.float32),
        pltpu.VMEM((2, D, 128), jnp.float32),
        pltpu.SemaphoreType.DMA,
        pltpu.SemaphoreType.DMA,
        pltpu.SemaphoreType.DMA,
        pltpu.SemaphoreType.DMA,
    ],
    compiler_params=_PARAMS,
)
def _gather(x_t, pairs, out, idx_all, gidx, hbuf, prows, tout, gg0, gg1, go0,
            go1):
  wid = lax.axis_index("s") * NC + lax.axis_index("c")
  gsems = [gg0, gg1]
  osems = [go0, go1]
  rowd = [_iota16() + dg * 16 for dg in range(4)]
  perms = [jnp.bitwise_and(_iota16() + k, 15) for k in range(16)]

  pltpu.sync_copy(x_t.at[:, pl.ds(wid * 128, 128)], idx_all)

  def prep_and_fire(t, buf):
    for k in range(8):
      v = idx_all[t, pl.ds(k * 16, 16)]
      gidx[buf, pl.ds(k * 16, 16)] = lax.shift_right_logical(v, 1)
      hbuf[buf, pl.ds(k * 16, 16)] = lax.shift_left(
          jnp.bitwise_and(v, 1), 6
      )
    pltpu.async_copy(pairs.at[gidx.at[buf]], prows.at[buf], gsems[buf])

  def drain_gather(buf):
    pltpu.make_async_copy(
        pairs.at[gidx.at[buf]], prows.at[buf], gsems[buf]
    ).wait()

  def drain_out(buf):
    pltpu.make_async_copy(
        tout.at[buf], out.at[0, :, pl.ds(0, 128)], osems[buf]
    ).wait()

  def transpose_block(buf, ob):
    src = prows.at[buf]
    dst = tout.at[ob]
    hsrc = hbuf.at[buf]

    @plsc.parallel_loop(0, 8, step=1, unroll=4)
    def _(bg):
      b0 = bg * 16
      for k in range(16):
        cold = perms[k] + b0
        halfp = plsc.load_gather(hsrc, [cold])
        vecs = [
            plsc.load_gather(src, [cold, halfp + rowd[dg]])
            for dg in range(4)
        ]
        for dg in range(4):
          plsc.store_scatter(dst, [rowd[dg], cold], vecs[dg])

  prep_and_fire(0, 0)

  def pair_step(p, carry):
    for half in range(2):
      i = 2 * p + half

      @pl.when(i + 1 < NT)
      def _():
        prep_and_fire(i + 1, 1 - half)

      drain_gather(half)

      @pl.when(i >= 2)
      def _():
        drain_out(half)

      transpose_block(half, half)
      pltpu.async_copy(
          tout.at[half], out.at[i, :, pl.ds(wid * 128, 128)], osems[half]
      )
    return carry

  lax.fori_loop(0, NT // 2, pair_step, 0)
  drain_out(0)
  drain_out(1)


@jax.jit
def kernel(x, table):
  B0, T = x.shape
  xt = jnp.asarray(x, jnp.int32).T
  tt = table.T
  pairs = _relayout(tt)
  tail = table[VT_FULL * 128 :, :].reshape(VT_TAIL // 2, 128)
  pairs = lax.dynamic_update_slice(pairs, tail, (VT_FULL * 64, 0))
  out5 = _gather(xt, pairs)
  return out5.transpose(2, 0, 1)
